# Initial kernel scaffold; baseline (speedup 1.0000x reference)
import jax
import jax.numpy as jnp
from jax.experimental import pallas as pl


def kernel(manifold_state, S_matrix, state_energy, W1, b1, W2, b2, running_mu):
    raise NotImplementedError("write your pallas kernel here")


# speedup vs baseline: 8.2154x; 8.2154x over previous
import jax
import jax.numpy as jnp
from jax.experimental import pallas as pl
from jax.experimental.pallas import tpu as pltpu

_B = 16384
_LD = 2048
_ED = 1024
_HID = 1024
_K = 32
_BLK = 256
_NBLK = _B // _BLK
_BLK2 = 512
_NBLK2 = _B // _BLK2
_DELTA_BASE = 0.01
_VOL_T = 0.5


def _bf16_round(x):
    u = jax.lax.bitcast_convert_type(x, jnp.int32)
    u = u + jnp.int32(0x7FFF) + ((u >> 16) & jnp.int32(1))
    u = u & jnp.int32(-65536)
    return jax.lax.bitcast_convert_type(u, jnp.float32)


def _vol_body(S_ref, E_ref, W1_ref, b1_ref, W2_ref, b2_ref, vol_ref, musum_ref):
    i = pl.program_id(0)

    x = S_ref[...]
    z = jax.lax.bitcast_convert_type(x, jnp.int32)
    z = z ^ ((z >> 31) & jnp.int32(0x7FFFFFFF))
    row_sum = jnp.sum(x, axis=1)

    cnt_nonneg = jnp.sum((z >= 0).astype(jnp.int32), axis=1)
    p = jnp.where(cnt_nonneg >= _K, jnp.int32(0), jnp.int32(-(2 ** 31)))
    for b in range(30, -1, -1):
        cand = p + jnp.int32(1 << b)
        cnt = jnp.sum((z >= cand[:, None]).astype(jnp.int32), axis=1)
        p = jnp.where(cnt >= _K, cand, p)

    gt = z > p[:, None]
    cnt_gt = jnp.sum(gt.astype(jnp.int32), axis=1)
    sum_gt = jnp.sum(jnp.where(gt, x, 0.0), axis=1)
    tz = p ^ ((p >> 31) & jnp.int32(0x7FFFFFFF))
    tval = jax.lax.bitcast_convert_type(tz, jnp.float32)
    topsum = sum_gt + (_K - cnt_gt).astype(jnp.float32) * tval
    eta = topsum / _K - row_sum / _LD

    e = E_ref[...]
    h = jax.lax.dot_general(e, W1_ref[...], (((1,), (0,)), ((), ())),
                            preferred_element_type=jnp.float32)
    h = jnp.maximum(h + b1_ref[...], 0.0)
    logit = jnp.sum(_bf16_round(h) * _bf16_round(W2_ref[...]), axis=1) + b2_ref[0]
    mu = jnp.maximum(logit, 0.0) + jnp.log1p(jnp.exp(-jnp.abs(logit)))

    vol_ref[...] = jnp.abs(eta) * mu

    @pl.when(i == 0)
    def _():
        musum_ref[0] = 0.0

    musum_ref[0] += jnp.sum(mu)


def _dream_body(scale_ref, vol_ref, M_ref, dream_ref):
    v = vol_ref[...]
    m = M_ref[...]
    dream_ref[...] = jnp.where(v[:, None] > _VOL_T, m * scale_ref[0], 0.0)


def kernel(manifold_state, S_matrix, state_energy, W1, b1, W2, b2, running_mu):
    vol, musum = pl.pallas_call(
        _vol_body,
        grid=(_NBLK,),
        in_specs=[
            pl.BlockSpec((_BLK, _LD), lambda i: (i, 0)),
            pl.BlockSpec((_BLK, _ED), lambda i: (i, 0)),
            pl.BlockSpec((_ED, _HID), lambda i: (0, 0)),
            pl.BlockSpec((1, _HID), lambda i: (0, 0)),
            pl.BlockSpec((1, _ED), lambda i: (0, 0)),
            pl.BlockSpec(memory_space=pltpu.SMEM),
        ],
        out_specs=[
            pl.BlockSpec((_BLK,), lambda i: (i,)),
            pl.BlockSpec(memory_space=pltpu.SMEM),
        ],
        out_shape=[
            jax.ShapeDtypeStruct((_B,), jnp.float32),
            jax.ShapeDtypeStruct((1,), jnp.float32),
        ],
    )(S_matrix, state_energy.astype(jnp.bfloat16), W1.astype(jnp.bfloat16),
      b1.reshape(1, _HID), W2.reshape(1, _ED), b2)

    mu_mean = musum[0] / _B
    new_running_mu = 0.9 * running_mu + 0.1 * mu_mean
    dynamic_delta = _DELTA_BASE * (1.0 + new_running_mu)
    scale = (1.0 + dynamic_delta).reshape(1)

    dream = pl.pallas_call(
        _dream_body,
        grid=(_NBLK2,),
        in_specs=[
            pl.BlockSpec(memory_space=pltpu.SMEM),
            pl.BlockSpec((_BLK2,), lambda i: (i,)),
            pl.BlockSpec((_BLK2, _LD), lambda i: (i, 0)),
        ],
        out_specs=pl.BlockSpec((_BLK2, _LD), lambda i: (i, 0)),
        out_shape=jax.ShapeDtypeStruct((_B, _LD), jnp.float32),
    )(scale, vol, manifold_state)

    return (vol, new_running_mu.reshape(()), dynamic_delta.reshape(()), dream)
